# Initial kernel scaffold; baseline (speedup 1.0000x reference)
import jax
import jax.numpy as jnp
from jax.experimental import pallas as pl


def kernel(xyz, x_tilde, nbrs, W_mlp, b_mlp, ab_mlp, G_s, G_p, G_d, P_1, P_2, D_1, D_2):
    raise NotImplementedError("write your pallas kernel here")


# speedup vs baseline: 14.3260x; 14.3260x over previous
import functools
from math import comb

import jax
import jax.numpy as jnp
import numpy as np
from jax import lax
from jax.experimental import pallas as pl
from jax.experimental.pallas import tpu as pltpu
from jax.experimental.pallas import tpu_sc as plsc

FEAT = 128
BERN_K = 16
GAMMA = 0.5
R_CUT = 5.0
N_NODES = 10000
N_EDGES = 160000

NC = 2
NS = 16
NW = NC * NS
CHUNK = 128
E_PER_W = 5120
E_PAD = NW * E_PER_W
N_CHUNKS = E_PER_W // CHUNK
ACC_N = 10112
TBL_W = 512

_PREC = lax.Precision.HIGHEST


def _swish(x, a, b):
    return a * x * jax.nn.sigmoid(b * x)


def _resmlp_block(x, W3, b3, ab6):
    h = _swish(x, ab6[0], ab6[1])
    h = lax.dot_general(h, W3[0], (((1,), (1,)), ((), ())),
                        preferred_element_type=jnp.float32, precision=_PREC)
    h = h + b3[0:1, :]
    h = _swish(h, ab6[2], ab6[3])
    h = lax.dot_general(h, W3[1], (((1,), (1,)), ((), ())),
                        preferred_element_type=jnp.float32, precision=_PREC)
    h = h + b3[1:2, :]
    h = x + h
    h = _swish(h, ab6[4], ab6[5])
    h = lax.dot_general(h, W3[2], (((1,), (1,)), ((), ())),
                        preferred_element_type=jnp.float32, precision=_PREC)
    return h + b3[2:3, :]


def _ab6(ab_ref, i):
    return [ab_ref[i * 6 + t] for t in range(6)]


def _node_mlp_body(x_ref, xyzp_ref, W_ref, b_ref, ab_ref, tbl_ref, c_ref):
    x = x_ref[...]
    c_ref[...] = _resmlp_block(x, W_ref[0], b_ref[0], _ab6(ab_ref, 0))
    for i in range(1, 4):
        rm = _resmlp_block(x, W_ref[i], b_ref[i], _ab6(ab_ref, i))
        tbl_ref[:, (i - 1) * FEAT:i * FEAT] = rm
    tbl_ref[:, 3 * FEAT:4 * FEAT] = xyzp_ref[...]


def _node_mlps(x_tilde, xyz_p, W_mlp, b_mlp, ab_flat):
    bn = 2000
    grid = (N_NODES // bn,)
    return pl.pallas_call(
        _node_mlp_body,
        grid=grid,
        in_specs=[
            pl.BlockSpec((bn, FEAT), lambda b: (b, 0)),
            pl.BlockSpec((bn, FEAT), lambda b: (b, 0)),
            pl.BlockSpec((5, 3, FEAT, FEAT), lambda b: (0, 0, 0, 0)),
            pl.BlockSpec((5, 3, FEAT), lambda b: (0, 0, 0)),
            pl.BlockSpec(memory_space=pltpu.SMEM),
        ],
        out_specs=[
            pl.BlockSpec((bn, TBL_W), lambda b: (b, 0)),
            pl.BlockSpec((bn, FEAT), lambda b: (b, 0)),
        ],
        out_shape=[
            jax.ShapeDtypeStruct((N_NODES, TBL_W), jnp.float32),
            jax.ShapeDtypeStruct((N_NODES, FEAT), jnp.float32),
        ],
    )(x_tilde, xyz_p, W_mlp, b_mlp, ab_flat)


def _make_gather(D):
    mesh = plsc.VectorSubcoreMesh(core_axis_name="c", subcore_axis_name="s",
                                  num_cores=NC, num_subcores=NS)

    def body(tbl_hbm, idx_hbm, out_hbm, idx_v, rows_v, sem):
        wid = lax.axis_index("s") * NC + lax.axis_index("c")
        pltpu.sync_copy(idx_hbm.at[wid], idx_v)

        def chunk_body(j, carry):
            pltpu.async_copy(tbl_hbm.at[idx_v.at[j]], rows_v, sem).wait()
            pltpu.sync_copy(rows_v, out_hbm.at[wid, j])
            return carry

        lax.fori_loop(0, N_CHUNKS, chunk_body, 0)

    return pl.kernel(
        body,
        out_type=jax.ShapeDtypeStruct((NW, N_CHUNKS, CHUNK, D), jnp.float32),
        mesh=mesh,
        scratch_types=[
            pltpu.VMEM((N_CHUNKS, CHUNK), jnp.int32),
            pltpu.VMEM((CHUNK, D), jnp.float32),
            pltpu.SemaphoreType.DMA,
        ],
    )


def _make_scatter(ch):
    mesh = plsc.VectorSubcoreMesh(core_axis_name="c", subcore_axis_name="s",
                                  num_cores=NC, num_subcores=NS)
    rows_per_s = ACC_N // NS

    def body(z_hbm, idx_hbm, zero_hbm, out_hbm, idx_v, z_v, acc, sem):
        c = lax.axis_index("c")
        s = lax.axis_index("s")
        wid = s * NC + c
        sl = pl.ds(s * rows_per_s, rows_per_s)
        pltpu.sync_copy(zero_hbm.at[sl], acc.at[sl])
        plsc.subcore_barrier()
        pltpu.sync_copy(idx_hbm.at[wid], idx_v)

        def chunk_body(j, carry):
            pltpu.async_copy(z_hbm.at[ch, wid, j], z_v, sem).wait()
            pltpu.sync_copy(z_v, acc.at[idx_v.at[j]], add=True)
            return carry

        lax.fori_loop(0, N_CHUNKS, chunk_body, 0)
        plsc.subcore_barrier()
        pltpu.sync_copy(acc.at[sl], out_hbm.at[c, sl])

    return pl.kernel(
        body,
        out_type=jax.ShapeDtypeStruct((NC, ACC_N, FEAT), jnp.float32),
        mesh=mesh,
        scratch_types=[
            pltpu.VMEM((N_CHUNKS, CHUNK), jnp.int32),
            pltpu.VMEM((CHUNK, FEAT), jnp.float32),
            pltpu.VMEM_SHARED((ACC_N, FEAT), jnp.float32),
            pltpu.SemaphoreType.DMA,
        ],
    )


def _edge_body(gs_ref, gd_ref, GT_ref, z_ref):
    g = gs_ref[...]
    rm_s = g[:, 0:FEAT]
    rm_p = g[:, FEAT:2 * FEAT]
    rm_d = g[:, 2 * FEAT:3 * FEAT]
    xyz_s = g[:, 3 * FEAT:4 * FEAT]
    r_full = xyz_s - gd_ref[...]
    r2 = jnp.sum(r_full * r_full, axis=1, keepdims=True) + 1e-12
    r = jnp.sqrt(r2)
    ex = jnp.exp(-GAMMA * r)
    one_m = 1.0 - ex

    pk = [jnp.ones_like(ex)]
    qk = [jnp.ones_like(ex)]
    for _ in range(BERN_K - 1):
        pk.append(pk[-1] * ex)
        qk.append(qk[-1] * one_m)

    denom = jnp.where(r < R_CUT, (R_CUT - r) * (R_CUT + r), 1.0)
    fcut = jnp.where(r < R_CUT, jnp.exp(-(r * r) / denom), 0.0)
    cols = [float(comb(BERN_K - 1, k)) * pk[k] * qk[BERN_K - 1 - k] * fcut
            for k in range(BERN_K)]
    rho = jnp.concatenate(cols, axis=1)

    r_safe = jnp.where(r > 1e-8, r, 1.0)
    u = r_full / r_safe
    ux = u[:, 0:1]
    uy = u[:, 1:2]
    uz = u[:, 2:3]

    h = [lax.dot_general(rho, GT_ref[l], (((1,), (0,)), ((), ())),
                         preferred_element_type=jnp.float32, precision=_PREC)
         for l in range(3)]

    z_ref[0] = rm_s * h[0]
    zp = rm_p * h[1]
    z_ref[1] = zp * ux
    z_ref[2] = zp * uy
    z_ref[3] = zp * uz
    zd = rm_d * h[2]
    z_ref[4] = zd * (ux * uy)
    z_ref[5] = zd * (uy * uz)
    z_ref[6] = zd * (3.0 * uz * uz - 1.0)
    z_ref[7] = zd * (ux * uz)
    z_ref[8] = zd * (ux * ux - uy * uy)


def _edge_z(g_src, g_dst, GT):
    be = 1024
    grid = (E_PAD // be,)
    return pl.pallas_call(
        _edge_body,
        grid=grid,
        in_specs=[
            pl.BlockSpec((be, TBL_W), lambda b: (b, 0)),
            pl.BlockSpec((be, FEAT), lambda b: (b, 0)),
            pl.BlockSpec((3, BERN_K, FEAT), lambda b: (0, 0, 0)),
        ],
        out_specs=pl.BlockSpec((9, be, FEAT), lambda b: (0, b, 0)),
        out_shape=jax.ShapeDtypeStruct((9, E_PAD, FEAT), jnp.float32),
    )(g_src, g_dst, GT)


def _combine_body(c_ref, q0, q1a, q1b, q1c, q2a, q2b, q2c, q2d, q2e,
                  P1_ref, P2_ref, D1_ref, D2_ref, W_ref, b_ref, ab_ref,
                  out_ref):
    acc = c_ref[...] + (q0[0] + q0[1])
    P1 = P1_ref[...]
    P2 = P2_ref[...]
    for qr in (q1a, q1b, q1c):
        q = qr[0] + qr[1]
        t1 = lax.dot_general(q, P1, (((1,), (1,)), ((), ())),
                             preferred_element_type=jnp.float32, precision=_PREC)
        t2 = lax.dot_general(q, P2, (((1,), (1,)), ((), ())),
                             preferred_element_type=jnp.float32, precision=_PREC)
        acc = acc + t1 * t2
    D1 = D1_ref[...]
    D2 = D2_ref[...]
    for qr in (q2a, q2b, q2c, q2d, q2e):
        q = qr[0] + qr[1]
        t1 = lax.dot_general(q, D1, (((1,), (1,)), ((), ())),
                             preferred_element_type=jnp.float32, precision=_PREC)
        t2 = lax.dot_general(q, D2, (((1,), (1,)), ((), ())),
                             preferred_element_type=jnp.float32, precision=_PREC)
        acc = acc + t1 * t2
    out_ref[...] = _resmlp_block(acc, W_ref[4], b_ref[4], _ab6(ab_ref, 4))


def _combine(c_term, scat, P_1, P_2, D_1, D_2, W_mlp, b_mlp, ab_flat):
    bn = 2000
    grid = (N_NODES // bn,)
    part_spec = pl.BlockSpec((NC, bn, FEAT), lambda b: (0, b, 0))
    return pl.pallas_call(
        _combine_body,
        grid=grid,
        in_specs=[pl.BlockSpec((bn, FEAT), lambda b: (b, 0))]
        + [part_spec] * 9
        + [pl.BlockSpec((FEAT, FEAT), lambda b: (0, 0))] * 4
        + [
            pl.BlockSpec((5, 3, FEAT, FEAT), lambda b: (0, 0, 0, 0)),
            pl.BlockSpec((5, 3, FEAT), lambda b: (0, 0, 0)),
            pl.BlockSpec(memory_space=pltpu.SMEM),
        ],
        out_specs=pl.BlockSpec((bn, FEAT), lambda b: (b, 0)),
        out_shape=jax.ShapeDtypeStruct((N_NODES, FEAT), jnp.float32),
    )(c_term, *scat, P_1, P_2, D_1, D_2, W_mlp, b_mlp, ab_flat)


def kernel(xyz, x_tilde, nbrs, W_mlp, b_mlp, ab_mlp, G_s, G_p, G_d,
           P_1, P_2, D_1, D_2):
    dst = nbrs[:, 0]
    src = nbrs[:, 1]

    xyz_p = jnp.zeros((N_NODES, FEAT), jnp.float32).at[:, 0:3].set(xyz)
    ab_flat = ab_mlp.reshape(30)
    GT = jnp.stack([G_s.T, G_p.T, G_d.T])

    tbl, c_term = _node_mlps(x_tilde, xyz_p, W_mlp, b_mlp, ab_flat)

    pad_e = E_PAD - N_EDGES
    src_p = jnp.concatenate([src, jnp.zeros((pad_e,), jnp.int32)])
    dst_p = jnp.concatenate(
        [dst, jnp.full((pad_e,), N_NODES, jnp.int32)])
    src_rs = src_p.reshape(NW, N_CHUNKS, CHUNK)
    dst_rs = dst_p.reshape(NW, N_CHUNKS, CHUNK)

    g_src = _make_gather(TBL_W)(tbl, src_rs).reshape(E_PAD, TBL_W)
    g_dst = _make_gather(FEAT)(xyz_p, dst_rs).reshape(E_PAD, FEAT)

    z = _edge_z(g_src, g_dst, GT)
    z_rs = z.reshape(9, NW, N_CHUNKS, CHUNK, FEAT)

    zero_acc = jnp.zeros((ACC_N, FEAT), jnp.float32)
    scat = [_make_scatter(ch)(z_rs, dst_rs, zero_acc) for ch in range(9)]

    return _combine(c_term, scat, P_1, P_2, D_1, D_2, W_mlp, b_mlp, ab_flat)


# speedup vs baseline: 16.4916x; 1.1512x over previous
import functools
from math import comb

import jax
import jax.numpy as jnp
import numpy as np
from jax import lax
from jax.experimental import pallas as pl
from jax.experimental.pallas import tpu as pltpu
from jax.experimental.pallas import tpu_sc as plsc

FEAT = 128
BERN_K = 16
GAMMA = 0.5
R_CUT = 5.0
N_NODES = 10000
N_EDGES = 160000

NC = 2
NS = 16
NW = NC * NS
E_PER_W = 5120
E_PAD = NW * E_PER_W
CG = 64
NG = E_PER_W // CG
CS = 128
NSCH = E_PER_W // CS
ACC_N = 10112
TBL_W = 512
XW = 128

_PREC = lax.Precision.HIGHEST


def _swish(x, a, b):
    return a * x * jax.nn.sigmoid(b * x)


def _resmlp_block(x, W3, b3, ab6):
    h = _swish(x, ab6[0], ab6[1])
    h = lax.dot_general(h, W3[0], (((1,), (1,)), ((), ())),
                        preferred_element_type=jnp.float32, precision=_PREC)
    h = h + b3[0:1, :]
    h = _swish(h, ab6[2], ab6[3])
    h = lax.dot_general(h, W3[1], (((1,), (1,)), ((), ())),
                        preferred_element_type=jnp.float32, precision=_PREC)
    h = h + b3[1:2, :]
    h = x + h
    h = _swish(h, ab6[4], ab6[5])
    h = lax.dot_general(h, W3[2], (((1,), (1,)), ((), ())),
                        preferred_element_type=jnp.float32, precision=_PREC)
    return h + b3[2:3, :]


def _ab6(ab_ref, i):
    return [ab_ref[i * 6 + t] for t in range(6)]


def _node_mlp_body(x_ref, xyz16_ref, W_ref, b_ref, ab_ref, tbl_ref, c_ref):
    x = x_ref[...]
    c_ref[...] = _resmlp_block(x, W_ref[0], b_ref[0], _ab6(ab_ref, 0))
    for i in range(1, 4):
        rm = _resmlp_block(x, W_ref[i], b_ref[i], _ab6(ab_ref, i))
        tbl_ref[:, (i - 1) * FEAT:i * FEAT] = rm
    tbl_ref[:, 3 * FEAT:4 * FEAT] = xyz16_ref[...]


def _node_mlps(x_tilde, xyz16, W_mlp, b_mlp, ab_flat):
    bn = 2000
    grid = (N_NODES // bn,)
    return pl.pallas_call(
        _node_mlp_body,
        grid=grid,
        in_specs=[
            pl.BlockSpec((bn, FEAT), lambda b: (b, 0)),
            pl.BlockSpec((bn, XW), lambda b: (b, 0)),
            pl.BlockSpec((5, 3, FEAT, FEAT), lambda b: (0, 0, 0, 0)),
            pl.BlockSpec((5, 3, FEAT), lambda b: (0, 0, 0)),
            pl.BlockSpec(memory_space=pltpu.SMEM),
        ],
        out_specs=[
            pl.BlockSpec((bn, TBL_W), lambda b: (b, 0)),
            pl.BlockSpec((bn, FEAT), lambda b: (b, 0)),
        ],
        out_shape=[
            jax.ShapeDtypeStruct((N_NODES, TBL_W), jnp.float32),
            jax.ShapeDtypeStruct((N_NODES, FEAT), jnp.float32),
        ],
    )(x_tilde, xyz16, W_mlp, b_mlp, ab_flat)


def _make_gather():
    mesh = plsc.VectorSubcoreMesh(core_axis_name="c", subcore_axis_name="s",
                                  num_cores=NC, num_subcores=NS)

    def body(tbl_hbm, xyz_hbm, idxs_hbm, idxd_hbm, outA_hbm, outB_hbm,
             idxs_v, idxd_v, bufA, bufB, semA0, semA1, semB0, semB1):
        wid = lax.axis_index("s") * NC + lax.axis_index("c")
        pltpu.sync_copy(idxs_hbm.at[wid], idxs_v)
        pltpu.sync_copy(idxd_hbm.at[wid], idxd_v)

        def pair(jp, carry):
            j0 = 2 * jp
            j1 = j0 + 1
            dA0 = pltpu.async_copy(tbl_hbm.at[idxs_v.at[j0]], bufA.at[0], semA0)
            dA1 = pltpu.async_copy(tbl_hbm.at[idxs_v.at[j1]], bufA.at[1], semA1)
            dB0 = pltpu.async_copy(xyz_hbm.at[idxd_v.at[j0]], bufB.at[0], semB0)
            dB1 = pltpu.async_copy(xyz_hbm.at[idxd_v.at[j1]], bufB.at[1], semB1)
            dA0.wait()
            pltpu.sync_copy(bufA.at[0], outA_hbm.at[wid, j0])
            dB0.wait()
            pltpu.sync_copy(bufB.at[0], outB_hbm.at[wid, j0])
            dA1.wait()
            pltpu.sync_copy(bufA.at[1], outA_hbm.at[wid, j1])
            dB1.wait()
            pltpu.sync_copy(bufB.at[1], outB_hbm.at[wid, j1])
            return carry

        lax.fori_loop(0, NG // 2, pair, 0)

    return pl.kernel(
        body,
        out_type=[
            jax.ShapeDtypeStruct((NW, NG, CG, TBL_W), jnp.float32),
            jax.ShapeDtypeStruct((NW, NG, CG, XW), jnp.float32),
        ],
        mesh=mesh,
        scratch_types=[
            pltpu.VMEM((NG, CG), jnp.int32),
            pltpu.VMEM((NG, CG), jnp.int32),
            pltpu.VMEM((2, CG, TBL_W), jnp.float32),
            pltpu.VMEM((2, CG, XW), jnp.float32),
            pltpu.SemaphoreType.DMA,
            pltpu.SemaphoreType.DMA,
            pltpu.SemaphoreType.DMA,
            pltpu.SemaphoreType.DMA,
        ],
    )


def _make_scatter():
    mesh = plsc.VectorSubcoreMesh(core_axis_name="c", subcore_axis_name="s",
                                  num_cores=NC, num_subcores=NS)
    rows_per_s = ACC_N // NS

    def body(z_hbm, idx_hbm, zero_hbm, out_hbm, idx_v, zbuf, acc,
             sem0, sem1):
        c = lax.axis_index("c")
        s = lax.axis_index("s")
        wid = s * NC + c
        sl = pl.ds(s * rows_per_s, rows_per_s)
        pltpu.sync_copy(idx_hbm.at[wid], idx_v)
        for ch in range(9):
            pltpu.sync_copy(zero_hbm.at[sl], acc.at[sl])
            plsc.subcore_barrier()

            def pair(jp, carry):
                j0 = 2 * jp
                j1 = j0 + 1
                d0 = pltpu.async_copy(z_hbm.at[ch, wid, j0], zbuf.at[0], sem0)
                d1 = pltpu.async_copy(z_hbm.at[ch, wid, j1], zbuf.at[1], sem1)
                d0.wait()
                pltpu.sync_copy(zbuf.at[0], acc.at[idx_v.at[j0]], add=True)
                d1.wait()
                pltpu.sync_copy(zbuf.at[1], acc.at[idx_v.at[j1]], add=True)
                return carry

            lax.fori_loop(0, NSCH // 2, pair, 0)
            plsc.subcore_barrier()
            pltpu.sync_copy(acc.at[sl], out_hbm.at[ch, c, sl])

    return pl.kernel(
        body,
        out_type=jax.ShapeDtypeStruct((9, NC, ACC_N, FEAT), jnp.float32),
        mesh=mesh,
        scratch_types=[
            pltpu.VMEM((NSCH, CS), jnp.int32),
            pltpu.VMEM((2, CS, FEAT), jnp.float32),
            pltpu.VMEM_SHARED((ACC_N, FEAT), jnp.float32),
            pltpu.SemaphoreType.DMA,
            pltpu.SemaphoreType.DMA,
        ],
    )


def _edge_body(gs_ref, gd_ref, GT_ref, z_ref):
    g = gs_ref[...]
    rm_s = g[:, 0:FEAT]
    rm_p = g[:, FEAT:2 * FEAT]
    rm_d = g[:, 2 * FEAT:3 * FEAT]
    xyz_s = g[:, 3 * FEAT:4 * FEAT]
    r16 = xyz_s - gd_ref[...]
    r2 = jnp.sum(r16 * r16, axis=1, keepdims=True) + 1e-12
    r = jnp.sqrt(r2)
    ex = jnp.exp(-GAMMA * r)
    one_m = 1.0 - ex

    pk = [jnp.ones_like(ex)]
    qk = [jnp.ones_like(ex)]
    for _ in range(BERN_K - 1):
        pk.append(pk[-1] * ex)
        qk.append(qk[-1] * one_m)

    denom = jnp.where(r < R_CUT, (R_CUT - r) * (R_CUT + r), 1.0)
    fcut = jnp.where(r < R_CUT, jnp.exp(-(r * r) / denom), 0.0)
    cols = [float(comb(BERN_K - 1, k)) * pk[k] * qk[BERN_K - 1 - k] * fcut
            for k in range(BERN_K)]
    rho = jnp.concatenate(cols, axis=1)

    r_safe = jnp.where(r > 1e-8, r, 1.0)
    u = r16 / r_safe
    ux = u[:, 0:1]
    uy = u[:, 1:2]
    uz = u[:, 2:3]

    h = [lax.dot_general(rho, GT_ref[l], (((1,), (0,)), ((), ())),
                         preferred_element_type=jnp.float32, precision=_PREC)
         for l in range(3)]

    z_ref[0] = rm_s * h[0]
    zp = rm_p * h[1]
    z_ref[1] = zp * ux
    z_ref[2] = zp * uy
    z_ref[3] = zp * uz
    zd = rm_d * h[2]
    z_ref[4] = zd * (ux * uy)
    z_ref[5] = zd * (uy * uz)
    z_ref[6] = zd * (3.0 * uz * uz - 1.0)
    z_ref[7] = zd * (ux * uz)
    z_ref[8] = zd * (ux * ux - uy * uy)


def _edge_z(g_src, g_dst, GT):
    be = 1024
    grid = (E_PAD // be,)
    return pl.pallas_call(
        _edge_body,
        grid=grid,
        in_specs=[
            pl.BlockSpec((be, TBL_W), lambda b: (b, 0)),
            pl.BlockSpec((be, XW), lambda b: (b, 0)),
            pl.BlockSpec((3, BERN_K, FEAT), lambda b: (0, 0, 0)),
        ],
        out_specs=pl.BlockSpec((9, be, FEAT), lambda b: (0, b, 0)),
        out_shape=jax.ShapeDtypeStruct((9, E_PAD, FEAT), jnp.float32),
    )(g_src, g_dst, GT)


def _combine_body(c_ref, scat_ref, P1_ref, P2_ref, D1_ref, D2_ref,
                  W_ref, b_ref, ab_ref, out_ref):
    acc = c_ref[...] + (scat_ref[0, 0] + scat_ref[0, 1])
    P1 = P1_ref[...]
    P2 = P2_ref[...]
    for ch in range(1, 4):
        q = scat_ref[ch, 0] + scat_ref[ch, 1]
        t1 = lax.dot_general(q, P1, (((1,), (1,)), ((), ())),
                             preferred_element_type=jnp.float32, precision=_PREC)
        t2 = lax.dot_general(q, P2, (((1,), (1,)), ((), ())),
                             preferred_element_type=jnp.float32, precision=_PREC)
        acc = acc + t1 * t2
    D1 = D1_ref[...]
    D2 = D2_ref[...]
    for ch in range(4, 9):
        q = scat_ref[ch, 0] + scat_ref[ch, 1]
        t1 = lax.dot_general(q, D1, (((1,), (1,)), ((), ())),
                             preferred_element_type=jnp.float32, precision=_PREC)
        t2 = lax.dot_general(q, D2, (((1,), (1,)), ((), ())),
                             preferred_element_type=jnp.float32, precision=_PREC)
        acc = acc + t1 * t2
    out_ref[...] = _resmlp_block(acc, W_ref[4], b_ref[4], _ab6(ab_ref, 4))


def _combine(c_term, scat, P_1, P_2, D_1, D_2, W_mlp, b_mlp, ab_flat):
    bn = 2000
    grid = (N_NODES // bn,)
    return pl.pallas_call(
        _combine_body,
        grid=grid,
        in_specs=[
            pl.BlockSpec((bn, FEAT), lambda b: (b, 0)),
            pl.BlockSpec((9, NC, bn, FEAT), lambda b: (0, 0, b, 0)),
        ]
        + [pl.BlockSpec((FEAT, FEAT), lambda b: (0, 0))] * 4
        + [
            pl.BlockSpec((5, 3, FEAT, FEAT), lambda b: (0, 0, 0, 0)),
            pl.BlockSpec((5, 3, FEAT), lambda b: (0, 0, 0)),
            pl.BlockSpec(memory_space=pltpu.SMEM),
        ],
        out_specs=pl.BlockSpec((bn, FEAT), lambda b: (b, 0)),
        out_shape=jax.ShapeDtypeStruct((N_NODES, FEAT), jnp.float32),
    )(c_term, scat, P_1, P_2, D_1, D_2, W_mlp, b_mlp, ab_flat)


def kernel(xyz, x_tilde, nbrs, W_mlp, b_mlp, ab_mlp, G_s, G_p, G_d,
           P_1, P_2, D_1, D_2):
    dst = nbrs[:, 0]
    src = nbrs[:, 1]

    xyz16 = jnp.zeros((N_NODES, XW), jnp.float32).at[:, 0:3].set(xyz)
    ab_flat = ab_mlp.reshape(30)
    GT = jnp.stack([G_s.T, G_p.T, G_d.T])

    tbl, c_term = _node_mlps(x_tilde, xyz16, W_mlp, b_mlp, ab_flat)

    pad_e = E_PAD - N_EDGES
    src_p = jnp.concatenate([src, jnp.zeros((pad_e,), jnp.int32)])
    dst_p = jnp.concatenate([dst, jnp.full((pad_e,), N_NODES, jnp.int32)])
    src_g = src_p.reshape(NW, NG, CG)
    dst_g = dst_p.reshape(NW, NG, CG)
    dst_s = dst_p.reshape(NW, NSCH, CS)

    g_src, g_dst = _make_gather()(tbl, xyz16, src_g, dst_g)
    g_src = g_src.reshape(E_PAD, TBL_W)
    g_dst = g_dst.reshape(E_PAD, XW)

    z = _edge_z(g_src, g_dst, GT)
    z_rs = z.reshape(9, NW, NSCH, CS, FEAT)

    zero_acc = jnp.zeros((ACC_N, FEAT), jnp.float32)
    scat = _make_scatter()(z_rs, dst_s, zero_acc)

    return _combine(c_term, scat, P_1, P_2, D_1, D_2, W_mlp, b_mlp, ab_flat)


# speedup vs baseline: 18.8054x; 1.1403x over previous
import functools
from math import comb

import jax
import jax.numpy as jnp
import numpy as np
from jax import lax
from jax.experimental import pallas as pl
from jax.experimental.pallas import tpu as pltpu
from jax.experimental.pallas import tpu_sc as plsc

FEAT = 128
BERN_K = 16
GAMMA = 0.5
R_CUT = 5.0
N_NODES = 10000
N_EDGES = 160000

NC = 2
NS = 16
NW = NC * NS
E_PER_W = 5120
E_PAD = NW * E_PER_W
CG = 64
NG = E_PER_W // CG
CS = 128
NSCH = E_PER_W // CS
ACC_N = 10112
E_HALF = E_PAD // 2
NG_H = NG // 2
NSCH_H = NSCH // 2
TBL_W = 512
XW = 128

_PREC = lax.Precision.HIGHEST


def _swish(x, a, b):
    return a * x * jax.nn.sigmoid(b * x)


def _resmlp_block(x, W3, b3, ab6):
    h = _swish(x, ab6[0], ab6[1])
    h = lax.dot_general(h, W3[0], (((1,), (1,)), ((), ())),
                        preferred_element_type=jnp.float32, precision=_PREC)
    h = h + b3[0:1, :]
    h = _swish(h, ab6[2], ab6[3])
    h = lax.dot_general(h, W3[1], (((1,), (1,)), ((), ())),
                        preferred_element_type=jnp.float32, precision=_PREC)
    h = h + b3[1:2, :]
    h = x + h
    h = _swish(h, ab6[4], ab6[5])
    h = lax.dot_general(h, W3[2], (((1,), (1,)), ((), ())),
                        preferred_element_type=jnp.float32, precision=_PREC)
    return h + b3[2:3, :]


def _ab6(ab_ref, i):
    return [ab_ref[i * 6 + t] for t in range(6)]


def _node_mlp_body(x_ref, xyz16_ref, W_ref, b_ref, ab_ref, tbl_ref, c_ref):
    x = x_ref[...]
    c_ref[...] = _resmlp_block(x, W_ref[0], b_ref[0], _ab6(ab_ref, 0))
    for i in range(1, 4):
        rm = _resmlp_block(x, W_ref[i], b_ref[i], _ab6(ab_ref, i))
        tbl_ref[:, (i - 1) * FEAT:i * FEAT] = rm
    tbl_ref[:, 3 * FEAT:4 * FEAT] = xyz16_ref[...]


def _node_mlps(x_tilde, xyz16, W_mlp, b_mlp, ab_flat):
    bn = 2000
    grid = (N_NODES // bn,)
    return pl.pallas_call(
        _node_mlp_body,
        grid=grid,
        in_specs=[
            pl.BlockSpec((bn, FEAT), lambda b: (b, 0)),
            pl.BlockSpec((bn, XW), lambda b: (b, 0)),
            pl.BlockSpec((5, 3, FEAT, FEAT), lambda b: (0, 0, 0, 0)),
            pl.BlockSpec((5, 3, FEAT), lambda b: (0, 0, 0)),
            pl.BlockSpec(memory_space=pltpu.SMEM),
        ],
        out_specs=[
            pl.BlockSpec((bn, TBL_W), lambda b: (b, 0)),
            pl.BlockSpec((bn, FEAT), lambda b: (b, 0)),
        ],
        out_shape=[
            jax.ShapeDtypeStruct((N_NODES, TBL_W), jnp.float32),
            jax.ShapeDtypeStruct((N_NODES, FEAT), jnp.float32),
        ],
    )(x_tilde, xyz16, W_mlp, b_mlp, ab_flat)


def _make_gather(ng):
    mesh = plsc.VectorSubcoreMesh(core_axis_name="c", subcore_axis_name="s",
                                  num_cores=NC, num_subcores=NS)

    def body(tbl_hbm, xyz_hbm, idxs_hbm, idxd_hbm, outA_hbm, outB_hbm,
             idxs_v, idxd_v, bufA, bufB, semA0, semA1, semB0, semB1):
        wid = lax.axis_index("s") * NC + lax.axis_index("c")
        pltpu.sync_copy(idxs_hbm.at[wid], idxs_v)
        pltpu.sync_copy(idxd_hbm.at[wid], idxd_v)

        def pair(jp, carry):
            j0 = 2 * jp
            j1 = j0 + 1
            dA0 = pltpu.async_copy(tbl_hbm.at[idxs_v.at[j0]], bufA.at[0], semA0)
            dA1 = pltpu.async_copy(tbl_hbm.at[idxs_v.at[j1]], bufA.at[1], semA1)
            dB0 = pltpu.async_copy(xyz_hbm.at[idxd_v.at[j0]], bufB.at[0], semB0)
            dB1 = pltpu.async_copy(xyz_hbm.at[idxd_v.at[j1]], bufB.at[1], semB1)
            dA0.wait()
            pltpu.sync_copy(bufA.at[0], outA_hbm.at[wid, j0])
            dB0.wait()
            pltpu.sync_copy(bufB.at[0], outB_hbm.at[wid, j0])
            dA1.wait()
            pltpu.sync_copy(bufA.at[1], outA_hbm.at[wid, j1])
            dB1.wait()
            pltpu.sync_copy(bufB.at[1], outB_hbm.at[wid, j1])
            return carry

        lax.fori_loop(0, ng // 2, pair, 0)

    return pl.kernel(
        body,
        out_type=[
            jax.ShapeDtypeStruct((NW, ng, CG, TBL_W), jnp.float32),
            jax.ShapeDtypeStruct((NW, ng, CG, XW), jnp.float32),
        ],
        mesh=mesh,
        scratch_types=[
            pltpu.VMEM((ng, CG), jnp.int32),
            pltpu.VMEM((ng, CG), jnp.int32),
            pltpu.VMEM((2, CG, TBL_W), jnp.float32),
            pltpu.VMEM((2, CG, XW), jnp.float32),
            pltpu.SemaphoreType.DMA,
            pltpu.SemaphoreType.DMA,
            pltpu.SemaphoreType.DMA,
            pltpu.SemaphoreType.DMA,
        ],
    )


def _make_scatter():
    mesh = plsc.VectorSubcoreMesh(core_axis_name="c", subcore_axis_name="s",
                                  num_cores=NC, num_subcores=NS)
    rows_per_s = ACC_N // NS

    def body(z0_hbm, z1_hbm, idx0_hbm, idx1_hbm, zero_hbm, out_hbm,
             idx0_v, idx1_v, zbuf, acc, sem0, sem1):
        c = lax.axis_index("c")
        s = lax.axis_index("s")
        wid = s * NC + c
        sl = pl.ds(s * rows_per_s, rows_per_s)
        pltpu.sync_copy(idx0_hbm.at[wid], idx0_v)
        pltpu.sync_copy(idx1_hbm.at[wid], idx1_v)
        for ch in range(9):
            pltpu.sync_copy(zero_hbm.at[sl], acc.at[sl])
            plsc.subcore_barrier()

            for z_hbm, idx_v in ((z0_hbm, idx0_v), (z1_hbm, idx1_v)):
                def pair(jp, carry, z_hbm=z_hbm, idx_v=idx_v):
                    j0 = 2 * jp
                    j1 = j0 + 1
                    d0 = pltpu.async_copy(z_hbm.at[ch, wid, j0], zbuf.at[0], sem0)
                    d1 = pltpu.async_copy(z_hbm.at[ch, wid, j1], zbuf.at[1], sem1)
                    d0.wait()
                    pltpu.sync_copy(zbuf.at[0], acc.at[idx_v.at[j0]], add=True)
                    d1.wait()
                    pltpu.sync_copy(zbuf.at[1], acc.at[idx_v.at[j1]], add=True)
                    return carry

                lax.fori_loop(0, NSCH_H // 2, pair, 0)
            plsc.subcore_barrier()
            pltpu.sync_copy(acc.at[sl], out_hbm.at[ch, c, sl])

    return pl.kernel(
        body,
        out_type=jax.ShapeDtypeStruct((9, NC, ACC_N, FEAT), jnp.float32),
        mesh=mesh,
        scratch_types=[
            pltpu.VMEM((NSCH_H, CS), jnp.int32),
            pltpu.VMEM((NSCH_H, CS), jnp.int32),
            pltpu.VMEM((2, CS, FEAT), jnp.float32),
            pltpu.VMEM_SHARED((ACC_N, FEAT), jnp.float32),
            pltpu.SemaphoreType.DMA,
            pltpu.SemaphoreType.DMA,
        ],
    )


def _edge_body(gs_ref, gd_ref, GT_ref, z_ref):
    g = gs_ref[...]
    rm_s = g[:, 0:FEAT]
    rm_p = g[:, FEAT:2 * FEAT]
    rm_d = g[:, 2 * FEAT:3 * FEAT]
    xyz_s = g[:, 3 * FEAT:4 * FEAT]
    r16 = xyz_s - gd_ref[...]
    r2 = jnp.sum(r16 * r16, axis=1, keepdims=True) + 1e-12
    r = jnp.sqrt(r2)
    ex = jnp.exp(-GAMMA * r)
    one_m = 1.0 - ex

    pk = [jnp.ones_like(ex)]
    qk = [jnp.ones_like(ex)]
    for _ in range(BERN_K - 1):
        pk.append(pk[-1] * ex)
        qk.append(qk[-1] * one_m)

    denom = jnp.where(r < R_CUT, (R_CUT - r) * (R_CUT + r), 1.0)
    fcut = jnp.where(r < R_CUT, jnp.exp(-(r * r) / denom), 0.0)
    cols = [float(comb(BERN_K - 1, k)) * pk[k] * qk[BERN_K - 1 - k] * fcut
            for k in range(BERN_K)]
    rho = jnp.concatenate(cols, axis=1)

    r_safe = jnp.where(r > 1e-8, r, 1.0)
    u = r16 / r_safe
    ux = u[:, 0:1]
    uy = u[:, 1:2]
    uz = u[:, 2:3]

    h = [lax.dot_general(rho, GT_ref[l], (((1,), (0,)), ((), ())),
                         preferred_element_type=jnp.float32, precision=_PREC)
         for l in range(3)]

    z_ref[0] = rm_s * h[0]
    zp = rm_p * h[1]
    z_ref[1] = zp * ux
    z_ref[2] = zp * uy
    z_ref[3] = zp * uz
    zd = rm_d * h[2]
    z_ref[4] = zd * (ux * uy)
    z_ref[5] = zd * (uy * uz)
    z_ref[6] = zd * (3.0 * uz * uz - 1.0)
    z_ref[7] = zd * (ux * uz)
    z_ref[8] = zd * (ux * ux - uy * uy)


def _edge_z(g_src, g_dst, GT, n_edges):
    be = 1024
    grid = (n_edges // be,)
    return pl.pallas_call(
        _edge_body,
        grid=grid,
        in_specs=[
            pl.BlockSpec((be, TBL_W), lambda b: (b, 0)),
            pl.BlockSpec((be, XW), lambda b: (b, 0)),
            pl.BlockSpec((3, BERN_K, FEAT), lambda b: (0, 0, 0)),
        ],
        out_specs=pl.BlockSpec((9, be, FEAT), lambda b: (0, b, 0)),
        out_shape=jax.ShapeDtypeStruct((9, n_edges, FEAT), jnp.float32),
    )(g_src, g_dst, GT)


def _combine_body(c_ref, scat_ref, P1_ref, P2_ref, D1_ref, D2_ref,
                  W_ref, b_ref, ab_ref, out_ref):
    acc = c_ref[...] + (scat_ref[0, 0] + scat_ref[0, 1])
    P1 = P1_ref[...]
    P2 = P2_ref[...]
    for ch in range(1, 4):
        q = scat_ref[ch, 0] + scat_ref[ch, 1]
        t1 = lax.dot_general(q, P1, (((1,), (1,)), ((), ())),
                             preferred_element_type=jnp.float32, precision=_PREC)
        t2 = lax.dot_general(q, P2, (((1,), (1,)), ((), ())),
                             preferred_element_type=jnp.float32, precision=_PREC)
        acc = acc + t1 * t2
    D1 = D1_ref[...]
    D2 = D2_ref[...]
    for ch in range(4, 9):
        q = scat_ref[ch, 0] + scat_ref[ch, 1]
        t1 = lax.dot_general(q, D1, (((1,), (1,)), ((), ())),
                             preferred_element_type=jnp.float32, precision=_PREC)
        t2 = lax.dot_general(q, D2, (((1,), (1,)), ((), ())),
                             preferred_element_type=jnp.float32, precision=_PREC)
        acc = acc + t1 * t2
    out_ref[...] = _resmlp_block(acc, W_ref[4], b_ref[4], _ab6(ab_ref, 4))


def _combine(c_term, scat, P_1, P_2, D_1, D_2, W_mlp, b_mlp, ab_flat):
    bn = 2000
    grid = (N_NODES // bn,)
    return pl.pallas_call(
        _combine_body,
        grid=grid,
        in_specs=[
            pl.BlockSpec((bn, FEAT), lambda b: (b, 0)),
            pl.BlockSpec((9, NC, bn, FEAT), lambda b: (0, 0, b, 0)),
        ]
        + [pl.BlockSpec((FEAT, FEAT), lambda b: (0, 0))] * 4
        + [
            pl.BlockSpec((5, 3, FEAT, FEAT), lambda b: (0, 0, 0, 0)),
            pl.BlockSpec((5, 3, FEAT), lambda b: (0, 0, 0)),
            pl.BlockSpec(memory_space=pltpu.SMEM),
        ],
        out_specs=pl.BlockSpec((bn, FEAT), lambda b: (b, 0)),
        out_shape=jax.ShapeDtypeStruct((N_NODES, FEAT), jnp.float32),
    )(c_term, scat, P_1, P_2, D_1, D_2, W_mlp, b_mlp, ab_flat)


def kernel(xyz, x_tilde, nbrs, W_mlp, b_mlp, ab_mlp, G_s, G_p, G_d,
           P_1, P_2, D_1, D_2):
    dst = nbrs[:, 0]
    src = nbrs[:, 1]

    xyz16 = jnp.zeros((N_NODES, XW), jnp.float32).at[:, 0:3].set(xyz)
    ab_flat = ab_mlp.reshape(30)
    GT = jnp.stack([G_s.T, G_p.T, G_d.T])

    tbl, c_term = _node_mlps(x_tilde, xyz16, W_mlp, b_mlp, ab_flat)

    pad_e = E_PAD - N_EDGES
    src_p = jnp.concatenate([src, jnp.zeros((pad_e,), jnp.int32)])
    dst_p = jnp.concatenate([dst, jnp.full((pad_e,), N_NODES, jnp.int32)])

    gather = _make_gather(NG_H)
    zs = []
    dst_ss = []
    for h in range(2):
        half = slice(h * E_HALF, (h + 1) * E_HALF)
        src_g = src_p[half].reshape(NW, NG_H, CG)
        dst_g = dst_p[half].reshape(NW, NG_H, CG)
        dst_ss.append(dst_p[half].reshape(NW, NSCH_H, CS))
        g_src, g_dst = gather(tbl, xyz16, src_g, dst_g)
        z = _edge_z(g_src.reshape(E_HALF, TBL_W), g_dst.reshape(E_HALF, XW),
                    GT, E_HALF)
        zs.append(z.reshape(9, NW, NSCH_H, CS, FEAT))

    zero_acc = jnp.zeros((ACC_N, FEAT), jnp.float32)
    scat = _make_scatter()(zs[0], zs[1], dst_ss[0], dst_ss[1], zero_acc)

    return _combine(c_term, scat, P_1, P_2, D_1, D_2, W_mlp, b_mlp, ab_flat)


# speedup vs baseline: 18.8202x; 1.0008x over previous
import functools
from math import comb

import jax
import jax.numpy as jnp
import numpy as np
from jax import lax
from jax.experimental import pallas as pl
from jax.experimental.pallas import tpu as pltpu
from jax.experimental.pallas import tpu_sc as plsc

FEAT = 128
BERN_K = 16
GAMMA = 0.5
R_CUT = 5.0
N_NODES = 10000
N_EDGES = 160000

NC = 2
NS = 16
NW = NC * NS
E_PER_W = 5120
E_PAD = NW * E_PER_W
CG = 64
NG = E_PER_W // CG
CS = 128
NSCH = E_PER_W // CS
ACC_N = 10112
E_HALF = E_PAD // 2
NG_H = NG // 2
NSCH_H = NSCH // 2
TBL_W = 512
XW = 128

_PREC = lax.Precision.HIGHEST


def _swish(x, a, b):
    return a * x * jax.nn.sigmoid(b * x)


def _resmlp_block(x, W3, b3, ab6):
    h = _swish(x, ab6[0], ab6[1])
    h = lax.dot_general(h, W3[0], (((1,), (1,)), ((), ())),
                        preferred_element_type=jnp.float32, precision=_PREC)
    h = h + b3[0:1, :]
    h = _swish(h, ab6[2], ab6[3])
    h = lax.dot_general(h, W3[1], (((1,), (1,)), ((), ())),
                        preferred_element_type=jnp.float32, precision=_PREC)
    h = h + b3[1:2, :]
    h = x + h
    h = _swish(h, ab6[4], ab6[5])
    h = lax.dot_general(h, W3[2], (((1,), (1,)), ((), ())),
                        preferred_element_type=jnp.float32, precision=_PREC)
    return h + b3[2:3, :]


def _ab6(ab_ref, i):
    return [ab_ref[i * 6 + t] for t in range(6)]


def _node_mlp_body(x_ref, xyz16_ref, W_ref, b_ref, ab_ref, tbl_ref, c_ref):
    x = x_ref[...]
    c_ref[...] = _resmlp_block(x, W_ref[0], b_ref[0], _ab6(ab_ref, 0))
    for i in range(1, 4):
        rm = _resmlp_block(x, W_ref[i], b_ref[i], _ab6(ab_ref, i))
        tbl_ref[:, (i - 1) * FEAT:i * FEAT] = rm
    tbl_ref[:, 3 * FEAT:4 * FEAT] = xyz16_ref[...]


def _node_mlps(x_tilde, xyz16, W_mlp, b_mlp, ab_flat):
    bn = 2000
    grid = (N_NODES // bn,)
    return pl.pallas_call(
        _node_mlp_body,
        grid=grid,
        in_specs=[
            pl.BlockSpec((bn, FEAT), lambda b: (b, 0)),
            pl.BlockSpec((bn, XW), lambda b: (b, 0)),
            pl.BlockSpec((5, 3, FEAT, FEAT), lambda b: (0, 0, 0, 0)),
            pl.BlockSpec((5, 3, FEAT), lambda b: (0, 0, 0)),
            pl.BlockSpec(memory_space=pltpu.SMEM),
        ],
        out_specs=[
            pl.BlockSpec((bn, TBL_W), lambda b: (b, 0)),
            pl.BlockSpec((bn, FEAT), lambda b: (b, 0)),
        ],
        out_shape=[
            jax.ShapeDtypeStruct((N_NODES, TBL_W), jnp.float32),
            jax.ShapeDtypeStruct((N_NODES, FEAT), jnp.float32),
        ],
    )(x_tilde, xyz16, W_mlp, b_mlp, ab_flat)


def _make_gather(ng):
    mesh = plsc.VectorSubcoreMesh(core_axis_name="c", subcore_axis_name="s",
                                  num_cores=NC, num_subcores=NS)

    def body(tbl_hbm, xyz_hbm, idxs_hbm, idxd_hbm, outA_hbm, outB_hbm,
             idxs_v, idxd_v, bufA, bufB, semA0, semA1, semB0, semB1):
        wid = lax.axis_index("s") * NC + lax.axis_index("c")
        pltpu.sync_copy(idxs_hbm.at[wid], idxs_v)
        pltpu.sync_copy(idxd_hbm.at[wid], idxd_v)

        def pair(jp, carry):
            j0 = 2 * jp
            j1 = j0 + 1
            dA0 = pltpu.async_copy(tbl_hbm.at[idxs_v.at[j0]], bufA.at[0], semA0)
            dA1 = pltpu.async_copy(tbl_hbm.at[idxs_v.at[j1]], bufA.at[1], semA1)
            dB0 = pltpu.async_copy(xyz_hbm.at[idxd_v.at[j0]], bufB.at[0], semB0)
            dB1 = pltpu.async_copy(xyz_hbm.at[idxd_v.at[j1]], bufB.at[1], semB1)
            dA0.wait()
            pltpu.sync_copy(bufA.at[0], outA_hbm.at[wid, j0])
            dB0.wait()
            pltpu.sync_copy(bufB.at[0], outB_hbm.at[wid, j0])
            dA1.wait()
            pltpu.sync_copy(bufA.at[1], outA_hbm.at[wid, j1])
            dB1.wait()
            pltpu.sync_copy(bufB.at[1], outB_hbm.at[wid, j1])
            return carry

        lax.fori_loop(0, ng // 2, pair, 0)

    return pl.kernel(
        body,
        out_type=[
            jax.ShapeDtypeStruct((NW, ng, CG, TBL_W), jnp.float32),
            jax.ShapeDtypeStruct((NW, ng, CG, XW), jnp.float32),
        ],
        mesh=mesh,
        scratch_types=[
            pltpu.VMEM((ng, CG), jnp.int32),
            pltpu.VMEM((ng, CG), jnp.int32),
            pltpu.VMEM((2, CG, TBL_W), jnp.float32),
            pltpu.VMEM((2, CG, XW), jnp.float32),
            pltpu.SemaphoreType.DMA,
            pltpu.SemaphoreType.DMA,
            pltpu.SemaphoreType.DMA,
            pltpu.SemaphoreType.DMA,
        ],
    )


def _make_scatter():
    mesh = plsc.VectorSubcoreMesh(core_axis_name="c", subcore_axis_name="s",
                                  num_cores=NC, num_subcores=NS)
    rows_per_s = ACC_N // NS

    def body(z0_hbm, z1_hbm, idx0_hbm, idx1_hbm, zero_hbm, out_hbm,
             idx0_v, idx1_v, zbuf, acc, *sems):
        c = lax.axis_index("c")
        s = lax.axis_index("s")
        wid = s * NC + c
        sl = pl.ds(s * rows_per_s, rows_per_s)
        pltpu.sync_copy(idx0_hbm.at[wid], idx0_v)
        pltpu.sync_copy(idx1_hbm.at[wid], idx1_v)
        for ch in range(9):
            pltpu.sync_copy(zero_hbm.at[sl], acc.at[sl])
            plsc.subcore_barrier()

            for z_hbm, idx_v in ((z0_hbm, idx0_v), (z1_hbm, idx1_v)):
                def pair(jp, carry, z_hbm=z_hbm, idx_v=idx_v):
                    j0 = 2 * jp
                    j1 = j0 + 1
                    d0 = pltpu.async_copy(z_hbm.at[ch, wid, j0], zbuf.at[0], sems[0])
                    d1 = pltpu.async_copy(z_hbm.at[ch, wid, j1], zbuf.at[1], sems[1])
                    d0.wait()
                    pltpu.sync_copy(zbuf.at[0], acc.at[idx_v.at[j0]], add=True)
                    d1.wait()
                    pltpu.sync_copy(zbuf.at[1], acc.at[idx_v.at[j1]], add=True)
                    return carry

                lax.fori_loop(0, NSCH_H // 2, pair, 0)
            plsc.subcore_barrier()
            pltpu.sync_copy(acc.at[sl], out_hbm.at[ch, c, sl])

    return pl.kernel(
        body,
        out_type=jax.ShapeDtypeStruct((9, NC, ACC_N, FEAT), jnp.float32),
        mesh=mesh,
        scratch_types=[
            pltpu.VMEM((NSCH_H, CS), jnp.int32),
            pltpu.VMEM((NSCH_H, CS), jnp.int32),
            pltpu.VMEM((2, CS, FEAT), jnp.float32),
            pltpu.VMEM_SHARED((ACC_N, FEAT), jnp.float32),
            pltpu.SemaphoreType.DMA,
            pltpu.SemaphoreType.DMA,
        ],
    )


def _edge_body(gs_ref, gd_ref, GT_ref, z_ref):
    g = gs_ref[...]
    rm_s = g[:, 0:FEAT]
    rm_p = g[:, FEAT:2 * FEAT]
    rm_d = g[:, 2 * FEAT:3 * FEAT]
    xyz_s = g[:, 3 * FEAT:4 * FEAT]
    r16 = xyz_s - gd_ref[...]
    r2 = jnp.sum(r16 * r16, axis=1, keepdims=True) + 1e-12
    r = jnp.sqrt(r2)
    ex = jnp.exp(-GAMMA * r)
    one_m = 1.0 - ex

    pk = [jnp.ones_like(ex)]
    qk = [jnp.ones_like(ex)]
    for _ in range(BERN_K - 1):
        pk.append(pk[-1] * ex)
        qk.append(qk[-1] * one_m)

    denom = jnp.where(r < R_CUT, (R_CUT - r) * (R_CUT + r), 1.0)
    fcut = jnp.where(r < R_CUT, jnp.exp(-(r * r) / denom), 0.0)
    cols = [float(comb(BERN_K - 1, k)) * pk[k] * qk[BERN_K - 1 - k] * fcut
            for k in range(BERN_K)]
    rho = jnp.concatenate(cols, axis=1)

    r_safe = jnp.where(r > 1e-8, r, 1.0)
    u = r16 / r_safe
    ux = u[:, 0:1]
    uy = u[:, 1:2]
    uz = u[:, 2:3]

    h = [lax.dot_general(rho, GT_ref[l], (((1,), (0,)), ((), ())),
                         preferred_element_type=jnp.float32, precision=_PREC)
         for l in range(3)]

    z_ref[0] = rm_s * h[0]
    zp = rm_p * h[1]
    z_ref[1] = zp * ux
    z_ref[2] = zp * uy
    z_ref[3] = zp * uz
    zd = rm_d * h[2]
    z_ref[4] = zd * (ux * uy)
    z_ref[5] = zd * (uy * uz)
    z_ref[6] = zd * (3.0 * uz * uz - 1.0)
    z_ref[7] = zd * (ux * uz)
    z_ref[8] = zd * (ux * ux - uy * uy)


def _edge_z(g_src, g_dst, GT, n_edges):
    be = 1024
    grid = (n_edges // be,)
    return pl.pallas_call(
        _edge_body,
        grid=grid,
        in_specs=[
            pl.BlockSpec((be, TBL_W), lambda b: (b, 0)),
            pl.BlockSpec((be, XW), lambda b: (b, 0)),
            pl.BlockSpec((3, BERN_K, FEAT), lambda b: (0, 0, 0)),
        ],
        out_specs=pl.BlockSpec((9, be, FEAT), lambda b: (0, b, 0)),
        out_shape=jax.ShapeDtypeStruct((9, n_edges, FEAT), jnp.float32),
    )(g_src, g_dst, GT)


def _combine_body(c_ref, scat_ref, P1_ref, P2_ref, D1_ref, D2_ref,
                  W_ref, b_ref, ab_ref, out_ref):
    acc = c_ref[...] + (scat_ref[0, 0] + scat_ref[0, 1])
    P1 = P1_ref[...]
    P2 = P2_ref[...]
    for ch in range(1, 4):
        q = scat_ref[ch, 0] + scat_ref[ch, 1]
        t1 = lax.dot_general(q, P1, (((1,), (1,)), ((), ())),
                             preferred_element_type=jnp.float32, precision=_PREC)
        t2 = lax.dot_general(q, P2, (((1,), (1,)), ((), ())),
                             preferred_element_type=jnp.float32, precision=_PREC)
        acc = acc + t1 * t2
    D1 = D1_ref[...]
    D2 = D2_ref[...]
    for ch in range(4, 9):
        q = scat_ref[ch, 0] + scat_ref[ch, 1]
        t1 = lax.dot_general(q, D1, (((1,), (1,)), ((), ())),
                             preferred_element_type=jnp.float32, precision=_PREC)
        t2 = lax.dot_general(q, D2, (((1,), (1,)), ((), ())),
                             preferred_element_type=jnp.float32, precision=_PREC)
        acc = acc + t1 * t2
    out_ref[...] = _resmlp_block(acc, W_ref[4], b_ref[4], _ab6(ab_ref, 4))


def _combine(c_term, scat, P_1, P_2, D_1, D_2, W_mlp, b_mlp, ab_flat):
    bn = 2000
    grid = (N_NODES // bn,)
    return pl.pallas_call(
        _combine_body,
        grid=grid,
        in_specs=[
            pl.BlockSpec((bn, FEAT), lambda b: (b, 0)),
            pl.BlockSpec((9, NC, bn, FEAT), lambda b: (0, 0, b, 0)),
        ]
        + [pl.BlockSpec((FEAT, FEAT), lambda b: (0, 0))] * 4
        + [
            pl.BlockSpec((5, 3, FEAT, FEAT), lambda b: (0, 0, 0, 0)),
            pl.BlockSpec((5, 3, FEAT), lambda b: (0, 0, 0)),
            pl.BlockSpec(memory_space=pltpu.SMEM),
        ],
        out_specs=pl.BlockSpec((bn, FEAT), lambda b: (b, 0)),
        out_shape=jax.ShapeDtypeStruct((N_NODES, FEAT), jnp.float32),
    )(c_term, scat, P_1, P_2, D_1, D_2, W_mlp, b_mlp, ab_flat)


def kernel(xyz, x_tilde, nbrs, W_mlp, b_mlp, ab_mlp, G_s, G_p, G_d,
           P_1, P_2, D_1, D_2):
    dst = nbrs[:, 0]
    src = nbrs[:, 1]

    xyz16 = jnp.zeros((N_NODES, XW), jnp.float32).at[:, 0:3].set(xyz)
    ab_flat = ab_mlp.reshape(30)
    GT = jnp.stack([G_s.T, G_p.T, G_d.T])

    tbl, c_term = _node_mlps(x_tilde, xyz16, W_mlp, b_mlp, ab_flat)

    pad_e = E_PAD - N_EDGES
    src_p = jnp.concatenate([src, jnp.zeros((pad_e,), jnp.int32)])
    dst_p = jnp.concatenate([dst, jnp.full((pad_e,), N_NODES, jnp.int32)])

    gather = _make_gather(NG_H)
    zs = []
    dst_ss = []
    for h in range(2):
        half = slice(h * E_HALF, (h + 1) * E_HALF)
        src_g = src_p[half].reshape(NW, NG_H, CG)
        dst_g = dst_p[half].reshape(NW, NG_H, CG)
        dst_ss.append(dst_p[half].reshape(NW, NSCH_H, CS))
        g_src, g_dst = gather(tbl, xyz16, src_g, dst_g)
        z = _edge_z(g_src.reshape(E_HALF, TBL_W), g_dst.reshape(E_HALF, XW),
                    GT, E_HALF)
        zs.append(z.reshape(9, NW, NSCH_H, CS, FEAT))

    zero_acc = jnp.zeros((ACC_N, FEAT), jnp.float32)
    scat = _make_scatter()(zs[0], zs[1], dst_ss[0], dst_ss[1], zero_acc)

    return _combine(c_term, scat, P_1, P_2, D_1, D_2, W_mlp, b_mlp, ab_flat)
